# Initial kernel scaffold; baseline (speedup 1.0000x reference)
import jax
import jax.numpy as jnp
from jax.experimental import pallas as pl


def kernel(x, embeddings):
    raise NotImplementedError("write your pallas kernel here")


# speedup vs baseline: 1.4698x; 1.4698x over previous
import functools

import jax
import jax.numpy as jnp
from jax import lax
from jax.experimental import pallas as pl
from jax.experimental.pallas import tpu as pltpu
from jax.experimental.pallas import tpu_sc as plsc

NUM_E = 8192
DIM = 32
ROWS = 64 * 1024
BLK = 256
NBLK = ROWS // BLK
LOSS_SCALE = 2.0 / (ROWS * DIM)


def _dist_argmin_body(x_ref, e_ref, idx_ref, loss_ref):
    x = x_ref[...]
    e = e_ref[...]
    esq = jnp.sum(e * e, axis=0, keepdims=True)
    xsq = jnp.sum(x * x, axis=1, keepdims=True)
    xb = x.astype(jnp.bfloat16)
    conv = jnp.dot(xb, e, preferred_element_type=jnp.float32)
    scores = (xsq + esq) - 2.0 * conv
    m = jnp.min(scores, axis=1, keepdims=True)
    col = lax.broadcasted_iota(jnp.int32, scores.shape, 1)
    idx = jnp.min(jnp.where(scores == m, col, NUM_E), axis=1)
    idx_ref[0, 0, :] = idx

    @pl.when(pl.program_id(0) == 0)
    def _init():
        loss_ref[0, 0] = 0.0

    loss_ref[0, 0] += jnp.sum(m) * LOSS_SCALE


def _dist_argmin(x2d, embeddings):
    return pl.pallas_call(
        _dist_argmin_body,
        grid=(NBLK,),
        in_specs=[
            pl.BlockSpec((BLK, DIM), lambda i: (i, 0)),
            pl.BlockSpec((DIM, NUM_E), lambda i: (0, 0)),
        ],
        out_specs=[
            pl.BlockSpec((1, 1, BLK), lambda i: (i, 0, 0)),
            pl.BlockSpec((1, 1), lambda i: (0, 0), memory_space=pltpu.SMEM),
        ],
        out_shape=[
            jax.ShapeDtypeStruct((NBLK, 1, BLK), jnp.int32),
            jax.ShapeDtypeStruct((1, 1), jnp.float32),
        ],
    )(x2d, embeddings)


def _make_sc_gather(n_workers, b_per_w):
    mesh = plsc.VectorSubcoreMesh(core_axis_name="c", subcore_axis_name="s")

    n_chunks = b_per_w // 128

    @functools.partial(
        pl.kernel,
        mesh=mesh,
        compiler_params=pltpu.CompilerParams(use_tc_tiling_on_sc=False),
        out_type=jax.ShapeDtypeStruct((ROWS, DIM), jnp.float32),
        scratch_types=[
            pltpu.VMEM((n_chunks, 128), jnp.int32),
            pltpu.VMEM((b_per_w, DIM), jnp.float32),
            pltpu.SemaphoreType.DMA,
        ],
    )
    def _sc_gather_body(table_hbm, idx_hbm, out_hbm, idx_v, rows_v, sem):
        n_cores = n_workers // 16
        wid = lax.axis_index("s") * n_cores + lax.axis_index("c")
        base = wid * b_per_w
        pltpu.sync_copy(idx_hbm.at[pl.ds(wid * n_chunks, n_chunks)], idx_v)
        copies = [
            pltpu.async_copy(
                table_hbm.at[idx_v.at[j]],
                rows_v.at[pl.ds(j * 128, 128), :],
                sem,
            )
            for j in range(n_chunks)
        ]
        for c in copies:
            c.wait()
        pltpu.sync_copy(rows_v, out_hbm.at[pl.ds(base, b_per_w)])

    return _sc_gather_body


def kernel(x, embeddings):
    x2d = x.reshape(ROWS, DIM)
    idx3, loss = _dist_argmin(x2d, embeddings)
    idx2d = idx3.reshape(ROWS // 128, 128)
    table = embeddings.T
    info = plsc.get_sparse_core_info()
    n_workers = info.num_cores * info.num_subcores
    gather = _make_sc_gather(n_workers, ROWS // n_workers)
    quantized = gather(table, idx2d)
    return quantized.reshape(x.shape), loss[0, 0]


# speedup vs baseline: 1.5355x; 1.0447x over previous
import functools

import jax
import jax.numpy as jnp
from jax import lax
from jax.experimental import pallas as pl
from jax.experimental.pallas import tpu as pltpu
from jax.experimental.pallas import tpu_sc as plsc

NUM_E = 8192
DIM = 32
ROWS = 64 * 1024
BLK = 512
NBLK = ROWS // BLK
LOSS_SCALE = 2.0 / (ROWS * DIM)


def _dist_argmin_body(x_ref, e_ref, idx_ref, loss_ref):
    x = x_ref[...]
    e = e_ref[...]
    esq = jnp.sum(e * e, axis=0, keepdims=True)
    xsq = jnp.sum(x * x, axis=1, keepdims=True)
    xb = x.astype(jnp.bfloat16)
    conv = jnp.dot(xb, e, preferred_element_type=jnp.float32)
    scores = (xsq + esq) - 2.0 * conv
    m = jnp.min(scores, axis=1, keepdims=True)
    col = lax.broadcasted_iota(jnp.int32, scores.shape, 1)
    idx = jnp.min(jnp.where(scores == m, col, NUM_E), axis=1)
    idx_ref[0, 0, :] = idx

    @pl.when(pl.program_id(0) == 0)
    def _init():
        loss_ref[0, 0] = 0.0

    loss_ref[0, 0] += jnp.sum(m) * LOSS_SCALE


def _dist_argmin(x2d, embeddings):
    return pl.pallas_call(
        _dist_argmin_body,
        grid=(NBLK,),
        in_specs=[
            pl.BlockSpec((BLK, DIM), lambda i: (i, 0)),
            pl.BlockSpec((DIM, NUM_E), lambda i: (0, 0)),
        ],
        out_specs=[
            pl.BlockSpec((1, 1, BLK), lambda i: (i, 0, 0)),
            pl.BlockSpec((1, 1), lambda i: (0, 0), memory_space=pltpu.SMEM),
        ],
        out_shape=[
            jax.ShapeDtypeStruct((NBLK, 1, BLK), jnp.int32),
            jax.ShapeDtypeStruct((1, 1), jnp.float32),
        ],
    )(x2d, embeddings)


def _make_sc_gather(n_workers, b_per_w):
    mesh = plsc.VectorSubcoreMesh(core_axis_name="c", subcore_axis_name="s")

    n_chunks = b_per_w // 128

    @functools.partial(
        pl.kernel,
        mesh=mesh,
        compiler_params=pltpu.CompilerParams(use_tc_tiling_on_sc=False),
        out_type=jax.ShapeDtypeStruct((ROWS, DIM), jnp.float32),
        scratch_types=[
            pltpu.VMEM((n_chunks, 128), jnp.int32),
            pltpu.VMEM((b_per_w, DIM), jnp.float32),
            pltpu.SemaphoreType.DMA,
        ],
    )
    def _sc_gather_body(table_hbm, idx_hbm, out_hbm, idx_v, rows_v, sem):
        n_cores = n_workers // 16
        wid = lax.axis_index("s") * n_cores + lax.axis_index("c")
        base = wid * b_per_w
        pltpu.sync_copy(idx_hbm.at[pl.ds(wid * n_chunks, n_chunks)], idx_v)
        copies = [
            pltpu.async_copy(
                table_hbm.at[idx_v.at[j]],
                rows_v.at[pl.ds(j * 128, 128), :],
                sem,
            )
            for j in range(n_chunks)
        ]
        for c in copies:
            c.wait()
        pltpu.sync_copy(rows_v, out_hbm.at[pl.ds(base, b_per_w)])

    return _sc_gather_body


def kernel(x, embeddings):
    x2d = x.reshape(ROWS, DIM)
    idx3, loss = _dist_argmin(x2d, embeddings)
    idx2d = idx3.reshape(ROWS // 128, 128)
    table = embeddings.T
    info = plsc.get_sparse_core_info()
    n_workers = info.num_cores * info.num_subcores
    gather = _make_sc_gather(n_workers, ROWS // n_workers)
    quantized = gather(table, idx2d)
    return quantized.reshape(x.shape), loss[0, 0]
